# baseline (speedup 1.0000x reference)
import functools

import jax
import jax.numpy as jnp
from jax import lax
from jax.experimental import pallas as pl
from jax.experimental.pallas import tpu as pltpu

T, D, H, E, A = 4096, 768, 64, 14, 128
TILE_T = 256
K = 3


def _moe_kernel(x_ref, wg1_ref, bg1_ref, wg2_ref, bg2_ref, wer_ref, be_ref,
                y_ref):
    x = x_ref[...]
    h = jnp.maximum(
        jnp.dot(x, wg1_ref[...], preferred_element_type=jnp.float32)
        + bg1_ref[...], 0.0)
    logits_t = lax.dot_general(
        wg2_ref[...], h, (((0,), (1,)), ((), ())),
        preferred_element_type=jnp.float32) + bg2_ref[...].T
    m = jnp.max(logits_t, axis=0, keepdims=True)
    ex = jnp.exp(logits_t - m)
    w = ex / jnp.sum(ex, axis=0, keepdims=True)

    row = lax.broadcasted_iota(jnp.int32, w.shape, 0)
    mask = jnp.zeros(w.shape, jnp.bool_)
    for _ in range(K):
        cand = jnp.where(mask, -1.0, w)
        mx = jnp.max(cand, axis=0, keepdims=True)
        first = jnp.min(jnp.where(cand == mx, row, E), axis=0, keepdims=True)
        mask = mask | (row == first)
    combine = jnp.where(mask, w, 0.0).T

    acc = jnp.dot(combine, be_ref[...], preferred_element_type=jnp.float32)
    for e in range(E):
        xe = jnp.dot(x, wer_ref[e], preferred_element_type=jnp.float32)
        acc = acc + combine[:, e:e + 1] * xe
    y_ref[...] = acc


@functools.partial(jax.jit, static_argnames=())
def _run(x, Wg1, bg1, Wg2, bg2, WeR, be):
    grid = (T // TILE_T,)
    return pl.pallas_call(
        _moe_kernel,
        grid=grid,
        in_specs=[
            pl.BlockSpec((TILE_T, D), lambda i: (i, 0)),
            pl.BlockSpec((D, H), lambda i: (0, 0)),
            pl.BlockSpec((1, H), lambda i: (0, 0)),
            pl.BlockSpec((H, E), lambda i: (0, 0)),
            pl.BlockSpec((1, E), lambda i: (0, 0)),
            pl.BlockSpec((E, D, A), lambda i: (0, 0, 0)),
            pl.BlockSpec((E, A), lambda i: (0, 0)),
        ],
        out_specs=pl.BlockSpec((TILE_T, A), lambda i: (i, 0)),
        out_shape=jax.ShapeDtypeStruct((T, A), jnp.float32),
    )(x, Wg1, bg1, Wg2, bg2, WeR, be)


def kernel(x, Wg1, bg1, Wg2, bg2, We, be, top_k):
    return _run(x, Wg1, bg1.reshape(1, H), Wg2, bg2.reshape(1, E), We, be)


# speedup vs baseline: 1.2483x; 1.2483x over previous
import functools

import jax
import jax.numpy as jnp
from jax import lax
from jax.experimental import pallas as pl
from jax.experimental.pallas import tpu as pltpu

T, D, H, E, A = 4096, 768, 64, 14, 128
TILE_T = 1024
K = 3


def _moe_kernel(x_ref, wg1_ref, bg1_ref, wg2_ref, bg2_ref, wer_ref, be_ref,
                y_ref):
    x = x_ref[...]
    h = jnp.maximum(
        jnp.dot(x, wg1_ref[...], preferred_element_type=jnp.float32)
        + bg1_ref[...], 0.0)
    logits_t = lax.dot_general(
        wg2_ref[...], h, (((0,), (1,)), ((), ())),
        preferred_element_type=jnp.float32) + bg2_ref[...].T
    m = jnp.max(logits_t, axis=0, keepdims=True)
    ex = jnp.exp(logits_t - m)
    w = ex / jnp.sum(ex, axis=0, keepdims=True)

    row = lax.broadcasted_iota(jnp.int32, w.shape, 0)
    mask = jnp.zeros(w.shape, jnp.bool_)
    for _ in range(K):
        cand = jnp.where(mask, -1.0, w)
        mx = jnp.max(cand, axis=0, keepdims=True)
        first = jnp.min(jnp.where(cand == mx, row, E), axis=0, keepdims=True)
        mask = mask | (row == first)
    combine = jnp.where(mask, w, 0.0).T

    acc = jnp.dot(combine, be_ref[...], preferred_element_type=jnp.float32)
    for e in range(E):
        xe = jnp.dot(x, wer_ref[e], preferred_element_type=jnp.float32)
        acc = acc + combine[:, e:e + 1] * xe
    y_ref[...] = acc


@functools.partial(jax.jit, static_argnames=())
def _run(x, Wg1, bg1, Wg2, bg2, WeR, be):
    grid = (T // TILE_T,)
    return pl.pallas_call(
        _moe_kernel,
        grid=grid,
        in_specs=[
            pl.BlockSpec((TILE_T, D), lambda i: (i, 0)),
            pl.BlockSpec((D, H), lambda i: (0, 0)),
            pl.BlockSpec((1, H), lambda i: (0, 0)),
            pl.BlockSpec((H, E), lambda i: (0, 0)),
            pl.BlockSpec((1, E), lambda i: (0, 0)),
            pl.BlockSpec((E, D, A), lambda i: (0, 0, 0)),
            pl.BlockSpec((E, A), lambda i: (0, 0)),
        ],
        out_specs=pl.BlockSpec((TILE_T, A), lambda i: (i, 0)),
        out_shape=jax.ShapeDtypeStruct((T, A), jnp.float32),
    )(x, Wg1, bg1, Wg2, bg2, WeR, be)


def kernel(x, Wg1, bg1, Wg2, bg2, We, be, top_k):
    return _run(x, Wg1, bg1.reshape(1, H), Wg2, bg2.reshape(1, E), We, be)
